# Initial kernel scaffold; baseline (speedup 1.0000x reference)
import jax
import jax.numpy as jnp
from jax.experimental import pallas as pl


def kernel(pc1_0, pc1_1, pc1_2, pc1_3, pc2_0, pc2_1, pc2_2, pc2_3, flow_0, flow_1, flow_2, flow_3):
    raise NotImplementedError("write your pallas kernel here")


# speedup vs baseline: 4.9679x; 4.9679x over previous
import functools

import jax
import jax.numpy as jnp
from jax import lax
from jax.experimental import pallas as pl

_B = 2
_ALPHA = [0.02, 0.04, 0.08, 0.16]
_CHAMFER_W = 1.0
_SMOOTH_W = 1.0
_CURVATURE_W = 0.3
_INF = 3.4e38
_HI = jax.lax.Precision.HIGHEST


def _blk(n):
    return {4096: 128, 2048: 256, 1024: 512, 512: 512}[n]


def _pairdist(rows, cols):
    mm = jnp.dot(rows.astype(jnp.bfloat16), cols.astype(jnp.bfloat16),
                 preferred_element_type=jnp.float32)
    s1 = rows[:, 0:1] ** 2 + rows[:, 1:2] ** 2 + rows[:, 2:3] ** 2
    s2 = cols[0:1, :] ** 2 + cols[1:2, :] ** 2 + cols[2:3, :] ** 2
    d = -2.0 * mm
    d = d + s1
    d = d + s2
    return d


def _extract_min(d, iota, n):
    minv = jnp.min(d, axis=1, keepdims=True)
    idx = jnp.min(jnp.where(d <= minv, iota, n), axis=1,
                  keepdims=True)
    sel = iota == idx
    return minv, sel


def _gather(sel, vals):
    return jnp.dot(sel.astype(jnp.float32), vals,
                   preferred_element_type=jnp.float32, precision=_HI)


def _self_curv_kernel(nblk, n, p_rows_ref, p_all_ref, pT_all_ref, curv_ref):
    rows = p_rows_ref[0]
    d = _pairdist(rows, p_all_ref[0])
    iota = lax.broadcasted_iota(jnp.int32, (nblk, n), 1)
    acc = jnp.zeros((nblk, 3), jnp.float32)
    for j in range(10):
        _, sel = _extract_min(d, iota, n)
        acc = acc + (_gather(sel, pT_all_ref[0]) - rows)
        if j < 9:
            d = jnp.where(sel, _INF, d)
    curv_ref[0] = acc / 9.0


def _smooth_curv_kernel(nblk, n, p_rows_ref, p_all_ref, fT_all_ref,
                        f_rows_ref, wT_all_ref, w_rows_ref,
                        smooth_ref, moved_ref):
    b = pl.program_id(0)
    r = pl.program_id(1)

    @pl.when((b == 0) & (r == 0))
    def _():
        smooth_ref[:, :] = jnp.zeros((1, 1), jnp.float32)

    rows = p_rows_ref[0]
    d = _pairdist(rows, p_all_ref[0])
    iota = lax.broadcasted_iota(jnp.int32, (nblk, n), 1)
    f_all = fT_all_ref[0]
    f_rows = f_rows_ref[0]
    w_all = wT_all_ref[0]
    w_rows = w_rows_ref[0]
    macc = jnp.zeros((nblk, 3), jnp.float32)
    sacc = jnp.zeros((nblk, 1), jnp.float32)
    for j in range(10):
        _, sel = _extract_min(d, iota, n)
        macc = macc + (_gather(sel, w_all) - w_rows)
        if j < 9:
            diff = _gather(sel, f_all) - f_rows
            sacc = sacc + jnp.sqrt(jnp.sum(diff * diff, axis=1,
                                           keepdims=True))
            d = jnp.where(sel, _INF, d)
    moved_ref[0] = macc / 9.0
    smooth_ref[:, :] += jnp.sum(sacc, keepdims=True) / 8.0


def _cross_kernel(nblk, nblocks, n, w_rows_ref, p2_all_ref, curv2_ref,
                  moved_ref, colmin_ref, cham_ref, curvl_ref):
    b = pl.program_id(0)
    r = pl.program_id(1)

    @pl.when((b == 0) & (r == 0))
    def _():
        cham_ref[:, :] = jnp.zeros((1, 1), jnp.float32)
        curvl_ref[:, :] = jnp.zeros((1, 1), jnp.float32)

    rows = w_rows_ref[0]
    d = _pairdist(rows, p2_all_ref[0])

    cm = jnp.min(d, axis=0, keepdims=True)

    @pl.when(r == 0)
    def _():
        colmin_ref[0] = cm

    @pl.when(r != 0)
    def _():
        colmin_ref[0] = jnp.minimum(colmin_ref[0], cm)

    iota = lax.broadcasted_iota(jnp.int32, (nblk, n), 1)
    curv2 = curv2_ref[0]
    invds = []
    gs = []
    dist1 = None
    for j in range(5):
        minv, sel = _extract_min(d, iota, n)
        if j == 0:
            dist1 = minv
        invds.append(1.0 / (minv + 1e-8))
        gs.append(_gather(sel, curv2))
        if j < 4:
            d = jnp.where(sel, _INF, d)
    norm = invds[0]
    for j in range(1, 5):
        norm = norm + invds[j]
    inter = (invds[0] / norm) * gs[0]
    for j in range(1, 5):
        inter = inter + (invds[j] / norm) * gs[j]
    dmo = inter - moved_ref[0]
    cham_ref[:, :] += jnp.sum(dist1, keepdims=True)
    curvl_ref[:, :] += jnp.sum(dmo * dmo, keepdims=True)

    @pl.when(r == nblocks - 1)
    def _():
        cham_ref[:, :] += jnp.sum(colmin_ref[0], keepdims=True)


def _scale_losses(p1, p2, flow, n):
    nblk = _blk(n)
    nblocks = n // nblk
    p1T = jnp.transpose(p1, (0, 2, 1))
    p2T = jnp.transpose(p2, (0, 2, 1))
    fT = jnp.transpose(flow, (0, 2, 1))
    wT = p1T + fT

    grid = (_B, nblocks)
    rows3 = pl.BlockSpec((1, nblk, 3), lambda b, r: (b, r, 0))
    full3T = pl.BlockSpec((1, n, 3), lambda b, r: (b, 0, 0))
    full3 = pl.BlockSpec((1, 3, n), lambda b, r: (b, 0, 0))
    scalar = pl.BlockSpec((1, 1), lambda b, r: (0, 0))

    curv2 = pl.pallas_call(
        functools.partial(_self_curv_kernel, nblk, n),
        grid=grid,
        in_specs=[rows3, full3, full3T],
        out_specs=rows3,
        out_shape=jax.ShapeDtypeStruct((_B, n, 3), jnp.float32),
    )(p2T, p2, p2T)

    smooth, moved = pl.pallas_call(
        functools.partial(_smooth_curv_kernel, nblk, n),
        grid=grid,
        in_specs=[rows3, full3, full3T, rows3, full3T, rows3],
        out_specs=[scalar, rows3],
        out_shape=[jax.ShapeDtypeStruct((1, 1), jnp.float32),
                   jax.ShapeDtypeStruct((_B, n, 3), jnp.float32)],
    )(p1T, p1, fT, fT, wT, wT)

    colspec = pl.BlockSpec((1, 1, n), lambda b, r: (b, 0, 0))
    _, cham, curvl = pl.pallas_call(
        functools.partial(_cross_kernel, nblk, nblocks, n),
        grid=grid,
        in_specs=[rows3, full3, full3T, rows3],
        out_specs=[colspec, scalar, scalar],
        out_shape=[jax.ShapeDtypeStruct((_B, 1, n), jnp.float32),
                   jax.ShapeDtypeStruct((1, 1), jnp.float32),
                   jax.ShapeDtypeStruct((1, 1), jnp.float32)],
    )(wT, p2, curv2, moved)

    return cham, smooth, curvl


def kernel(pc1_0, pc1_1, pc1_2, pc1_3, pc2_0, pc2_1, pc2_2, pc2_3,
           flow_0, flow_1, flow_2, flow_3):
    pc1s = [pc1_0, pc1_1, pc1_2, pc1_3]
    pc2s = [pc2_0, pc2_1, pc2_2, pc2_3]
    flows = [flow_0, flow_1, flow_2, flow_3]
    ns = [4096, 2048, 1024, 512]
    total = jnp.zeros((), jnp.float32)
    for i in range(4):
        cham, smooth, curvl = _scale_losses(pc1s[i], pc2s[i], flows[i], ns[i])
        per = (_CHAMFER_W * cham[0, 0] + _SMOOTH_W * smooth[0, 0]
               + _CURVATURE_W * curvl[0, 0])
        total = total + (_ALPHA[i] / _B) * per
    return total.reshape(1)


# speedup vs baseline: 6.2990x; 1.2679x over previous
import functools

import jax
import jax.numpy as jnp
from jax import lax
from jax.experimental import pallas as pl

_B = 2
_ALPHA = [0.02, 0.04, 0.08, 0.16]
_CHAMFER_W = 1.0
_SMOOTH_W = 1.0
_CURVATURE_W = 0.3
_INF = 3.4e38
_HI = jax.lax.Precision.HIGHEST


def _blk(n):
    return {4096: 256, 2048: 512, 1024: 1024, 512: 512}[n]


def _blk_a(n):
    return {4096: 128, 2048: 256, 1024: 512, 512: 512}[n]


def _pairdist(rows, cols):
    mm = jnp.dot(rows.astype(jnp.bfloat16), cols.astype(jnp.bfloat16),
                 preferred_element_type=jnp.float32)
    s1 = rows[:, 0:1] ** 2 + rows[:, 1:2] ** 2 + rows[:, 2:3] ** 2
    s2 = cols[0:1, :] ** 2 + cols[1:2, :] ** 2 + cols[2:3, :] ** 2
    d = -2.0 * mm
    d = d + s1
    d = d + s2
    return d


def _extract_min(d, iota, n):
    minv = jnp.min(d, axis=1, keepdims=True)
    idx = jnp.min(jnp.where(d <= minv, iota, n), axis=1,
                  keepdims=True)
    sel = iota == idx
    return minv, sel


def _gather(sel, vals):
    return jnp.dot(sel.astype(jnp.float32), vals,
                   preferred_element_type=jnp.float32, precision=_HI)


def _self_curv_kernel(nblk, n, p_rows_ref, p_all_ref, pT_all_ref, curv_ref):
    rows = p_rows_ref[0]
    d = _pairdist(rows, p_all_ref[0])
    iota = lax.broadcasted_iota(jnp.int32, (nblk, n), 1)
    acc = jnp.zeros((nblk, 3), jnp.float32)
    for j in range(10):
        _, sel = _extract_min(d, iota, n)
        acc = acc + (_gather(sel, pT_all_ref[0]) - rows)
        if j < 9:
            d = jnp.where(sel, _INF, d)
    curv_ref[0] = acc / 9.0


def _smooth_curv_kernel(nblk, n, p_rows_ref, p_all_ref, fT_all_ref,
                        f_rows_ref, wT_all_ref, w_rows_ref,
                        smooth_ref, moved_ref):
    b = pl.program_id(0)
    r = pl.program_id(1)

    @pl.when((b == 0) & (r == 0))
    def _():
        smooth_ref[:, :] = jnp.zeros((1, 1), jnp.float32)

    rows = p_rows_ref[0]
    d = _pairdist(rows, p_all_ref[0])
    iota = lax.broadcasted_iota(jnp.int32, (nblk, n), 1)
    f_all = fT_all_ref[0]
    f_rows = f_rows_ref[0]
    w_all = wT_all_ref[0]
    w_rows = w_rows_ref[0]
    selsum = jnp.zeros((nblk, n), jnp.float32)
    sacc = jnp.zeros((nblk, 1), jnp.float32)
    for j in range(10):
        _, sel = _extract_min(d, iota, n)
        selsum = selsum + sel.astype(jnp.float32)
        if j < 9:
            diff = _gather(sel, f_all) - f_rows
            sacc = sacc + jnp.sqrt(jnp.sum(diff * diff, axis=1,
                                           keepdims=True))
            d = jnp.where(sel, _INF, d)
    nbr = jnp.dot(selsum, w_all, preferred_element_type=jnp.float32,
                  precision=_HI)
    moved_ref[0] = (nbr - 10.0 * w_rows) / 9.0
    smooth_ref[:, :] += jnp.sum(sacc, keepdims=True) / 8.0


def _cross_kernel(nblk, nblocks, n, w_rows_ref, p2_all_ref, curv2_ref,
                  moved_ref, colmin_ref, cham_ref, curvl_ref):
    b = pl.program_id(0)
    r = pl.program_id(1)

    @pl.when((b == 0) & (r == 0))
    def _():
        cham_ref[:, :] = jnp.zeros((1, 1), jnp.float32)
        curvl_ref[:, :] = jnp.zeros((1, 1), jnp.float32)

    rows = w_rows_ref[0]
    d = _pairdist(rows, p2_all_ref[0])

    cm = jnp.min(d, axis=0, keepdims=True)

    @pl.when(r == 0)
    def _():
        colmin_ref[0] = cm

    @pl.when(r != 0)
    def _():
        colmin_ref[0] = jnp.minimum(colmin_ref[0], cm)

    iota = lax.broadcasted_iota(jnp.int32, (nblk, n), 1)
    curv2 = curv2_ref[0]
    invds = []
    gs = []
    dist1 = None
    for j in range(5):
        minv, sel = _extract_min(d, iota, n)
        if j == 0:
            dist1 = minv
        invds.append(1.0 / (minv + 1e-8))
        gs.append(_gather(sel, curv2))
        if j < 4:
            d = jnp.where(sel, _INF, d)
    norm = invds[0]
    for j in range(1, 5):
        norm = norm + invds[j]
    inter = (invds[0] / norm) * gs[0]
    for j in range(1, 5):
        inter = inter + (invds[j] / norm) * gs[j]
    dmo = inter - moved_ref[0]
    cham_ref[:, :] += jnp.sum(dist1, keepdims=True)
    curvl_ref[:, :] += jnp.sum(dmo * dmo, keepdims=True)

    @pl.when(r == nblocks - 1)
    def _():
        cham_ref[:, :] += jnp.sum(colmin_ref[0], keepdims=True)


def _scale_losses(p1, p2, flow, n):
    nblk = _blk(n)
    nblocks = n // nblk
    nblk_a = _blk_a(n)
    nblocks_a = n // nblk_a
    p1T = jnp.transpose(p1, (0, 2, 1))
    p2T = jnp.transpose(p2, (0, 2, 1))
    fT = jnp.transpose(flow, (0, 2, 1))
    wT = p1T + fT

    grid = (_B, nblocks)
    grid_a = (_B, nblocks_a)
    rows3 = pl.BlockSpec((1, nblk, 3), lambda b, r: (b, r, 0))
    rows3a = pl.BlockSpec((1, nblk_a, 3), lambda b, r: (b, r, 0))
    full3T = pl.BlockSpec((1, n, 3), lambda b, r: (b, 0, 0))
    full3 = pl.BlockSpec((1, 3, n), lambda b, r: (b, 0, 0))
    scalar = pl.BlockSpec((1, 1), lambda b, r: (0, 0))

    curv2 = pl.pallas_call(
        functools.partial(_self_curv_kernel, nblk, n),
        grid=grid,
        in_specs=[rows3, full3, full3T],
        out_specs=rows3,
        out_shape=jax.ShapeDtypeStruct((_B, n, 3), jnp.float32),
    )(p2T, p2, p2T)

    smooth, moved = pl.pallas_call(
        functools.partial(_smooth_curv_kernel, nblk_a, n),
        grid=grid_a,
        in_specs=[rows3a, full3, full3T, rows3a, full3T, rows3a],
        out_specs=[scalar, rows3a],
        out_shape=[jax.ShapeDtypeStruct((1, 1), jnp.float32),
                   jax.ShapeDtypeStruct((_B, n, 3), jnp.float32)],
    )(p1T, p1, fT, fT, wT, wT)

    colspec = pl.BlockSpec((1, 1, n), lambda b, r: (b, 0, 0))
    _, cham, curvl = pl.pallas_call(
        functools.partial(_cross_kernel, nblk, nblocks, n),
        grid=grid,
        in_specs=[rows3, full3, full3T, rows3],
        out_specs=[colspec, scalar, scalar],
        out_shape=[jax.ShapeDtypeStruct((_B, 1, n), jnp.float32),
                   jax.ShapeDtypeStruct((1, 1), jnp.float32),
                   jax.ShapeDtypeStruct((1, 1), jnp.float32)],
    )(wT, p2, curv2, moved)

    return cham, smooth, curvl


def kernel(pc1_0, pc1_1, pc1_2, pc1_3, pc2_0, pc2_1, pc2_2, pc2_3,
           flow_0, flow_1, flow_2, flow_3):
    pc1s = [pc1_0, pc1_1, pc1_2, pc1_3]
    pc2s = [pc2_0, pc2_1, pc2_2, pc2_3]
    flows = [flow_0, flow_1, flow_2, flow_3]
    ns = [4096, 2048, 1024, 512]
    total = jnp.zeros((), jnp.float32)
    for i in range(4):
        cham, smooth, curvl = _scale_losses(pc1s[i], pc2s[i], flows[i], ns[i])
        per = (_CHAMFER_W * cham[0, 0] + _SMOOTH_W * smooth[0, 0]
               + _CURVATURE_W * curvl[0, 0])
        total = total + (_ALPHA[i] / _B) * per
    return total.reshape(1)


# speedup vs baseline: 12.0973x; 1.9205x over previous
import functools

import jax
import jax.numpy as jnp
from jax import lax
from jax.experimental import pallas as pl

_B = 2
_ALPHA = [0.02, 0.04, 0.08, 0.16]
_CHAMFER_W = 1.0
_SMOOTH_W = 1.0
_CURVATURE_W = 0.3
_INF = 3.4e38
_HI = jax.lax.Precision.HIGHEST


def _blk(n):
    return {4096: 256, 2048: 512, 1024: 1024, 512: 512}[n]


def _blk_a(n):
    return {4096: 256, 2048: 512, 1024: 1024, 512: 512}[n]


def _pairdist(rows, cols):
    mm = jnp.dot(rows.astype(jnp.bfloat16), cols.astype(jnp.bfloat16),
                 preferred_element_type=jnp.float32)
    s1 = rows[:, 0:1] ** 2 + rows[:, 1:2] ** 2 + rows[:, 2:3] ** 2
    s2 = cols[0:1, :] ** 2 + cols[1:2, :] ** 2 + cols[2:3, :] ** 2
    d = -2.0 * mm
    d = d + s1
    d = d + s2
    return d


def _extract_min(d, iota, n):
    minv = jnp.min(d, axis=1, keepdims=True)
    idx = jnp.min(jnp.where(d <= minv, iota, n), axis=1,
                  keepdims=True)
    sel = iota == idx
    return minv, sel


def _gather(sel, vals):
    return jnp.dot(sel.astype(jnp.float32), vals,
                   preferred_element_type=jnp.float32, precision=_HI)


def _self_curv_kernel(nblk, n, p_rows_ref, p_all_ref, pT_all_ref, curv_ref):
    rows = p_rows_ref[0]
    d = _pairdist(rows, p_all_ref[0])
    iota = lax.broadcasted_iota(jnp.int32, (nblk, n), 1)
    mask = None
    for j in range(10):
        _, sel = _extract_min(d, iota, n)
        mask = sel if mask is None else mask | sel
        if j < 9:
            d = jnp.where(sel, _INF, d)
    nbr = jnp.dot(mask.astype(jnp.float32), pT_all_ref[0],
                  preferred_element_type=jnp.float32, precision=_HI)
    curv_ref[0] = (nbr - 10.0 * rows) / 9.0


def _smooth_curv_kernel(nblk, n, p_rows_ref, p_all_ref, f_all3_ref,
                        f_rows_ref, wT_all_ref, w_rows_ref,
                        smooth_ref, moved_ref):
    b = pl.program_id(0)
    r = pl.program_id(1)

    @pl.when((b == 0) & (r == 0))
    def _():
        smooth_ref[:, :] = jnp.zeros((1, 1), jnp.float32)

    rows = p_rows_ref[0]
    d = _pairdist(rows, p_all_ref[0])
    iota = lax.broadcasted_iota(jnp.int32, (nblk, n), 1)
    f_all3 = f_all3_ref[0]
    f_rows = f_rows_ref[0]
    w_all = wT_all_ref[0]
    w_rows = w_rows_ref[0]

    F = jnp.dot(f_rows, f_all3, preferred_element_type=jnp.float32,
                precision=_HI)
    qr = f_rows[:, 0:1] ** 2 + f_rows[:, 1:2] ** 2 + f_rows[:, 2:3] ** 2
    qc = f_all3[0:1, :] ** 2 + f_all3[1:2, :] ** 2 + f_all3[2:3, :] ** 2
    nt = jnp.sqrt(jnp.maximum(qr + qc - 2.0 * F, 0.0))

    mask = None
    mask9 = None
    for j in range(10):
        _, sel = _extract_min(d, iota, n)
        mask = sel if mask is None else mask | sel
        if j == 8:
            mask9 = mask
        if j < 9:
            d = jnp.where(sel, _INF, d)
    nbr = jnp.dot(mask.astype(jnp.float32), w_all,
                  preferred_element_type=jnp.float32, precision=_HI)
    moved_ref[0] = (nbr - 10.0 * w_rows) / 9.0
    srow = jnp.sum(jnp.where(mask9, nt, 0.0), axis=1, keepdims=True)
    smooth_ref[:, :] += jnp.sum(srow, keepdims=True) / 8.0


def _cross_kernel(nblk, nblocks, n, w_rows_ref, p2_all_ref, curv2_ref,
                  moved_ref, colmin_ref, cham_ref, curvl_ref):
    b = pl.program_id(0)
    r = pl.program_id(1)

    @pl.when((b == 0) & (r == 0))
    def _():
        cham_ref[:, :] = jnp.zeros((1, 1), jnp.float32)
        curvl_ref[:, :] = jnp.zeros((1, 1), jnp.float32)

    rows = w_rows_ref[0]
    d = _pairdist(rows, p2_all_ref[0])

    cm = jnp.min(d, axis=0, keepdims=True)

    @pl.when(r == 0)
    def _():
        colmin_ref[0] = cm

    @pl.when(r != 0)
    def _():
        colmin_ref[0] = jnp.minimum(colmin_ref[0], cm)

    iota = lax.broadcasted_iota(jnp.int32, (nblk, n), 1)
    curv2 = curv2_ref[0]
    invds = []
    gs = []
    dist1 = None
    for j in range(5):
        minv, sel = _extract_min(d, iota, n)
        if j == 0:
            dist1 = minv
        invds.append(1.0 / (minv + 1e-8))
        gs.append(_gather(sel, curv2))
        if j < 4:
            d = jnp.where(sel, _INF, d)
    norm = invds[0]
    for j in range(1, 5):
        norm = norm + invds[j]
    inter = (invds[0] / norm) * gs[0]
    for j in range(1, 5):
        inter = inter + (invds[j] / norm) * gs[j]
    dmo = inter - moved_ref[0]
    cham_ref[:, :] += jnp.sum(dist1, keepdims=True)
    curvl_ref[:, :] += jnp.sum(dmo * dmo, keepdims=True)

    @pl.when(r == nblocks - 1)
    def _():
        cham_ref[:, :] += jnp.sum(colmin_ref[0], keepdims=True)


def _scale_losses(p1, p2, flow, n):
    nblk = _blk(n)
    nblocks = n // nblk
    nblk_a = _blk_a(n)
    nblocks_a = n // nblk_a
    p1T = jnp.transpose(p1, (0, 2, 1))
    p2T = jnp.transpose(p2, (0, 2, 1))
    fT = jnp.transpose(flow, (0, 2, 1))
    wT = p1T + fT

    grid = (_B, nblocks)
    grid_a = (_B, nblocks_a)
    rows3 = pl.BlockSpec((1, nblk, 3), lambda b, r: (b, r, 0))
    rows3a = pl.BlockSpec((1, nblk_a, 3), lambda b, r: (b, r, 0))
    full3T = pl.BlockSpec((1, n, 3), lambda b, r: (b, 0, 0))
    full3 = pl.BlockSpec((1, 3, n), lambda b, r: (b, 0, 0))
    scalar = pl.BlockSpec((1, 1), lambda b, r: (0, 0))

    curv2 = pl.pallas_call(
        functools.partial(_self_curv_kernel, nblk, n),
        grid=grid,
        in_specs=[rows3, full3, full3T],
        out_specs=rows3,
        out_shape=jax.ShapeDtypeStruct((_B, n, 3), jnp.float32),
    )(p2T, p2, p2T)

    smooth, moved = pl.pallas_call(
        functools.partial(_smooth_curv_kernel, nblk_a, n),
        grid=grid_a,
        in_specs=[rows3a, full3, full3, rows3a, full3T, rows3a],
        out_specs=[scalar, rows3a],
        out_shape=[jax.ShapeDtypeStruct((1, 1), jnp.float32),
                   jax.ShapeDtypeStruct((_B, n, 3), jnp.float32)],
    )(p1T, p1, flow, fT, wT, wT)

    colspec = pl.BlockSpec((1, 1, n), lambda b, r: (b, 0, 0))
    _, cham, curvl = pl.pallas_call(
        functools.partial(_cross_kernel, nblk, nblocks, n),
        grid=grid,
        in_specs=[rows3, full3, full3T, rows3],
        out_specs=[colspec, scalar, scalar],
        out_shape=[jax.ShapeDtypeStruct((_B, 1, n), jnp.float32),
                   jax.ShapeDtypeStruct((1, 1), jnp.float32),
                   jax.ShapeDtypeStruct((1, 1), jnp.float32)],
    )(wT, p2, curv2, moved)

    return cham, smooth, curvl


def kernel(pc1_0, pc1_1, pc1_2, pc1_3, pc2_0, pc2_1, pc2_2, pc2_3,
           flow_0, flow_1, flow_2, flow_3):
    pc1s = [pc1_0, pc1_1, pc1_2, pc1_3]
    pc2s = [pc2_0, pc2_1, pc2_2, pc2_3]
    flows = [flow_0, flow_1, flow_2, flow_3]
    ns = [4096, 2048, 1024, 512]
    total = jnp.zeros((), jnp.float32)
    for i in range(4):
        cham, smooth, curvl = _scale_losses(pc1s[i], pc2s[i], flows[i], ns[i])
        per = (_CHAMFER_W * cham[0, 0] + _SMOOTH_W * smooth[0, 0]
               + _CURVATURE_W * curvl[0, 0])
        total = total + (_ALPHA[i] / _B) * per
    return total.reshape(1)
